# Initial kernel scaffold; baseline (speedup 1.0000x reference)
import jax
import jax.numpy as jnp
from jax.experimental import pallas as pl


def kernel(encoded_tokens, pos_table):
    raise NotImplementedError("write your pallas kernel here")


# speedup vs baseline: 1.4625x; 1.4625x over previous
import jax
import jax.numpy as jnp
from jax.experimental import pallas as pl


def _add_kernel(tok_ref, pos_ref, out_ref):
    out_ref[...] = tok_ref[...] + pos_ref[...]


def kernel(encoded_tokens, pos_table):
    B, T, D = encoded_tokens.shape
    TBLK = 256
    grid = (T // TBLK, B)
    return pl.pallas_call(
        _add_kernel,
        grid=grid,
        in_specs=[
            pl.BlockSpec((1, TBLK, D), lambda t, b: (b, t, 0)),
            pl.BlockSpec((TBLK, D), lambda t, b: (t, 0)),
        ],
        out_specs=pl.BlockSpec((1, TBLK, D), lambda t, b: (b, t, 0)),
        out_shape=jax.ShapeDtypeStruct((B, T, D), encoded_tokens.dtype),
    )(encoded_tokens, pos_table)
